# baseline (speedup 1.0000x reference)
import functools

import jax
import jax.numpy as jnp
from jax import lax
from jax.experimental import pallas as pl
from jax.experimental.pallas import tpu as pltpu
from jax.experimental.pallas import tpu_sc as plsc

D = 64
VOCAB = 25
NPAIR = 640
NW = 32
CHUNK = 128
K = 2


def _pair_table_body(emb_ref, pt_ref, wpt_ref, bp_ref, wjt_ref, bj_ref,
                     x_ref, out_ref, idx_ref):
    b, s = x_ref.shape
    x3 = x_ref[...].reshape(b, s // 128, 128)
    pid = x3[:, :, :D] * VOCAB + x3[:, :, D:]
    idx_ref[...] = pid.reshape((b * s) // 128, D)
    prop = jnp.dot(pt_ref[...], wpt_ref[...],
                   preferred_element_type=jnp.float32) + bp_ref[...]
    combined = jnp.concatenate([emb_ref[...], prop], axis=-1)
    fused = jnp.dot(combined, wjt_ref[...],
                    preferred_element_type=jnp.float32) + bj_ref[...]
    p = lax.broadcasted_iota(jnp.int32, (NPAIR, VOCAB), 0)
    v = lax.broadcasted_iota(jnp.int32, (NPAIR, VOCAB), 1)
    left = (p // VOCAB == v).astype(jnp.float32)
    right = (p % VOCAB == v).astype(jnp.float32)
    out_ref[...] = jnp.concatenate(
        [jnp.dot(left, fused, preferred_element_type=jnp.float32,
                 precision=lax.Precision.HIGHEST),
         jnp.dot(right, fused, preferred_element_type=jnp.float32,
                 precision=lax.Precision.HIGHEST)], axis=-1)


def _build_prologue(emb_table, property_table, Wp, bp, Wj, bj, x):
    b, s = x.shape
    return pl.pallas_call(
        _pair_table_body,
        out_shape=(jax.ShapeDtypeStruct((NPAIR, 2 * D), jnp.float32),
                   jax.ShapeDtypeStruct(((b * s) // 128, D), jnp.int32)),
    )(emb_table, property_table, Wp.T, bp.reshape(1, D), Wj.T,
      bj.reshape(1, D), x)


@functools.cache
def _make_gather(n_pair):
    per_w = n_pair // NW
    n_chunks = per_w // CHUNK
    n_phases = n_chunks // K
    mesh = plsc.VectorSubcoreMesh(core_axis_name="c", subcore_axis_name="s")

    @functools.partial(
        pl.kernel, mesh=mesh,
        compiler_params=pltpu.CompilerParams(use_tc_tiling_on_sc=True),
        out_type=jax.ShapeDtypeStruct((n_pair, 2 * D), jnp.float32),
        scratch_types=[
            pltpu.VMEM((n_chunks, CHUNK), jnp.int32),
            pltpu.VMEM((2, K, CHUNK, 2 * D), jnp.float32),
            pltpu.VMEM_SHARED((NPAIR, 2 * D), jnp.float32),
            pltpu.SemaphoreType.DMA,
            pltpu.SemaphoreType.DMA,
            pltpu.SemaphoreType.DMA,
            pltpu.SemaphoreType.DMA,
        ],
    )
    def gather(table_hbm, idx_hbm, out_hbm, idx_v, rows_v, table_sh,
               g0, g1, o0, o1):
        wid = lax.axis_index("s") * 2 + lax.axis_index("c")
        base = wid * per_w
        @pl.when(lax.axis_index("s") == 0)
        def _():
            pltpu.sync_copy(table_hbm, table_sh)

        pltpu.sync_copy(idx_hbm.at[pl.ds(wid * n_chunks, n_chunks), :], idx_v)
        plsc.subcore_barrier()
        gsems = (g0, g1)
        osems = (o0, o1)

        def g_copy(p, h, c):
            j = p * K + c
            return pltpu.make_async_copy(
                table_sh.at[idx_v.at[j]], rows_v.at[h].at[c], gsems[h])

        def o_copy(p, h, c):
            j = p * K + c
            return pltpu.make_async_copy(
                rows_v.at[h].at[c],
                out_hbm.at[pl.ds(base + j * CHUNK, CHUNK), :], osems[h])

        def fire_g(p, h):
            for c in range(K):
                g_copy(p, h, c).start()

        def wait_g(p, h):
            for c in range(K):
                g_copy(p, h, c).wait()

        def fire_o(p, h):
            for c in range(K):
                o_copy(p, h, c).start()

        def wait_o(p, h):
            for c in range(K):
                o_copy(p, h, c).wait()

        fire_g(0, 0)
        wait_g(0, 0)
        fire_o(0, 0)
        fire_g(1, 1)

        def body(i, carry):
            p0 = 2 * i + 1
            wait_g(p0, 1)
            fire_o(p0, 1)
            wait_o(p0 - 1, 0)
            fire_g(p0 + 1, 0)
            p1 = p0 + 1
            wait_g(p1, 0)
            fire_o(p1, 0)
            wait_o(p1 - 1, 1)
            fire_g(p1 + 1, 1)
            return carry

        lax.fori_loop(0, (n_phases - 2) // 2, body, 0)

        p = n_phases - 1
        wait_g(p, 1)
        fire_o(p, 1)
        wait_o(p - 1, 0)
        wait_o(p, 1)

    return gather


def _unpack_body(sel_ref, in_ref, out_ref):
    sel = sel_ref[...]

    def split(a):
        hi = a.astype(jnp.bfloat16)
        lo = (a - hi.astype(jnp.float32)).astype(jnp.bfloat16)
        return hi, lo

    for t in range(TILES_PER_CELL):
        blk = in_ref[pl.ds(t * D, D), :]
        a_hi, a_lo = split(blk[:, :D])
        b_hi, b_lo = split(blk[:, D:])
        m = jnp.concatenate([a_hi, a_lo, b_hi, b_lo], axis=0)
        out_ref[0, :, pl.ds(t * 128, 128)] = lax.dot_general(
            m, sel, (((0,), (0,)), ((), ())),
            preferred_element_type=jnp.float32)


TILES_PER_CELL = 8


def _unpack(pairs, b, s):
    eye = jnp.eye(D, dtype=jnp.bfloat16)
    zero = jnp.zeros((D, D), dtype=jnp.bfloat16)
    left = jnp.concatenate([eye, eye, zero, zero], axis=0)
    right = jnp.concatenate([zero, zero, eye, eye], axis=0)
    sel = jnp.concatenate([left, right], axis=1)
    scells = (s // 128) // TILES_PER_CELL
    return pl.pallas_call(
        _unpack_body,
        grid=(b, scells),
        compiler_params=pltpu.CompilerParams(
            fuse_transposed_lhs_in_matmul=True),
        in_specs=[pl.BlockSpec((4 * D, 2 * D), lambda i, j: (0, 0)),
                  pl.BlockSpec((TILES_PER_CELL * D, 2 * D),
                               lambda i, j: (i * scells + j, 0))],
        out_specs=pl.BlockSpec((1, D, TILES_PER_CELL * 128),
                               lambda i, j: (i, 0, j)),
        out_shape=jax.ShapeDtypeStruct((b, D, s), jnp.float32),
    )(sel, pairs)


def kernel(x, emb_table, Wp, bp, Wj, bj, property_table):
    b, s = x.shape
    n_pair = (b * s) // 2
    pair_table, pidx = _build_prologue(
        emb_table, property_table, Wp, bp, Wj, bj, x.astype(jnp.int32))
    pairs = _make_gather(n_pair)(pair_table, pidx.reshape(n_pair // 128, 128))
    out_t = _unpack(pairs, b, s)
    return out_t.transpose(0, 2, 1)


# speedup vs baseline: 1.3363x; 1.3363x over previous
import functools

import jax
import jax.numpy as jnp
from jax import lax
from jax.experimental import pallas as pl
from jax.experimental.pallas import tpu as pltpu
from jax.experimental.pallas import tpu_sc as plsc

D = 64
VOCAB = 25
NPAIR = 640
NW = 32
CHUNK = 128
K = 2


def _pair_table_body(emb_ref, pt_ref, wpt_ref, bp_ref, wjt_ref, bj_ref,
                     x_ref, out_ref, idx_ref):
    b, s = x_ref.shape
    x3 = x_ref[...].reshape(b, s // 128, 128)
    pid = x3[:, :, :D] * VOCAB + x3[:, :, D:]
    idx_ref[...] = pid.reshape((b * s) // 128, D)
    prop = jnp.dot(pt_ref[...], wpt_ref[...],
                   preferred_element_type=jnp.float32) + bp_ref[...]
    combined = jnp.concatenate([emb_ref[...], prop], axis=-1)
    fused = jnp.dot(combined, wjt_ref[...],
                    preferred_element_type=jnp.float32) + bj_ref[...]
    p = lax.broadcasted_iota(jnp.int32, (NPAIR, VOCAB), 0)
    v = lax.broadcasted_iota(jnp.int32, (NPAIR, VOCAB), 1)
    left = (p // VOCAB == v).astype(jnp.float32)
    right = (p % VOCAB == v).astype(jnp.float32)
    out_ref[...] = jnp.concatenate(
        [jnp.dot(left, fused, preferred_element_type=jnp.float32,
                 precision=lax.Precision.HIGHEST),
         jnp.dot(right, fused, preferred_element_type=jnp.float32,
                 precision=lax.Precision.HIGHEST)], axis=-1)


def _build_prologue(emb_table, property_table, Wp, bp, Wj, bj, x):
    b, s = x.shape
    return pl.pallas_call(
        _pair_table_body,
        out_shape=(jax.ShapeDtypeStruct((NPAIR, 2 * D), jnp.float32),
                   jax.ShapeDtypeStruct(((b * s) // 128, D), jnp.int32)),
    )(emb_table, property_table, Wp.T, bp.reshape(1, D), Wj.T,
      bj.reshape(1, D), x)


@functools.cache
def _make_gather(n_pair):
    per_w = n_pair // NW
    n_chunks = per_w // CHUNK
    n_phases = n_chunks // K
    mesh = plsc.VectorSubcoreMesh(core_axis_name="c", subcore_axis_name="s")

    @functools.partial(
        pl.kernel, mesh=mesh,
        compiler_params=pltpu.CompilerParams(use_tc_tiling_on_sc=True),
        out_type=jax.ShapeDtypeStruct((n_pair, 2 * D), jnp.float32),
        scratch_types=[
            pltpu.VMEM((n_chunks, CHUNK), jnp.int32),
            pltpu.VMEM((2, K, CHUNK, 2 * D), jnp.float32),
            pltpu.VMEM_SHARED((NPAIR, 2 * D), jnp.float32),
            pltpu.SemaphoreType.DMA,
            pltpu.SemaphoreType.DMA,
            pltpu.SemaphoreType.DMA,
            pltpu.SemaphoreType.DMA,
        ],
    )
    def gather(table_hbm, idx_hbm, out_hbm, idx_v, rows_v, table_sh,
               g0, g1, o0, o1):
        wid = lax.axis_index("s") * 2 + lax.axis_index("c")
        base = wid * per_w
        @pl.when(lax.axis_index("s") == 0)
        def _():
            pltpu.sync_copy(table_hbm, table_sh)

        pltpu.sync_copy(idx_hbm.at[pl.ds(wid * n_chunks, n_chunks), :], idx_v)
        plsc.subcore_barrier()
        gsems = (g0, g1)
        osems = (o0, o1)

        def g_copy(p, h, c):
            j = p * K + c
            return pltpu.make_async_copy(
                table_sh.at[idx_v.at[j]], rows_v.at[h].at[c], gsems[h])

        def o_copy(p, h, c):
            j = p * K + c
            return pltpu.make_async_copy(
                rows_v.at[h].at[c],
                out_hbm.at[pl.ds(base + j * CHUNK, CHUNK), :], osems[h])

        def fire_g(p, h):
            for c in range(K):
                g_copy(p, h, c).start()

        def wait_g(p, h):
            for c in range(K):
                g_copy(p, h, c).wait()

        def fire_o(p, h):
            for c in range(K):
                o_copy(p, h, c).start()

        def wait_o(p, h):
            for c in range(K):
                o_copy(p, h, c).wait()

        fire_g(0, 0)
        wait_g(0, 0)
        fire_o(0, 0)
        fire_g(1, 1)

        def body(i, carry):
            p0 = 2 * i + 1
            wait_g(p0, 1)
            fire_o(p0, 1)
            wait_o(p0 - 1, 0)
            fire_g(p0 + 1, 0)
            p1 = p0 + 1
            wait_g(p1, 0)
            fire_o(p1, 0)
            wait_o(p1 - 1, 1)
            fire_g(p1 + 1, 1)
            return carry

        lax.fori_loop(0, (n_phases - 2) // 2, body, 0)

        p = n_phases - 1
        wait_g(p, 1)
        fire_o(p, 1)
        wait_o(p - 1, 0)
        wait_o(p, 1)

    return gather


def _unpack_body(sel_ref, in_ref, out_ref):
    sel = sel_ref[...]

    def split(a):
        hi = a.astype(jnp.bfloat16)
        lo = (a - hi.astype(jnp.float32)).astype(jnp.bfloat16)
        return hi, lo

    for t in range(TILES_PER_CELL):
        blk = in_ref[pl.ds(t * D, D), :]
        a_hi, a_lo = split(blk[:, :D])
        b_hi, b_lo = split(blk[:, D:])
        m = jnp.concatenate([a_hi, a_lo, b_hi, b_lo], axis=0)
        out_ref[0, :, pl.ds(t * 128, 128)] = lax.dot_general(
            m, sel, (((0,), (0,)), ((), ())),
            preferred_element_type=jnp.float32)


TILES_PER_CELL = 16


def _unpack(pairs, b, s):
    eye = jnp.eye(D, dtype=jnp.bfloat16)
    zero = jnp.zeros((D, D), dtype=jnp.bfloat16)
    left = jnp.concatenate([eye, eye, zero, zero], axis=0)
    right = jnp.concatenate([zero, zero, eye, eye], axis=0)
    sel = jnp.concatenate([left, right], axis=1)
    scells = (s // 128) // TILES_PER_CELL
    return pl.pallas_call(
        _unpack_body,
        grid=(b, scells),
        compiler_params=pltpu.CompilerParams(
            fuse_transposed_lhs_in_matmul=True),
        in_specs=[pl.BlockSpec((4 * D, 2 * D), lambda i, j: (0, 0)),
                  pl.BlockSpec((TILES_PER_CELL * D, 2 * D),
                               lambda i, j: (i * scells + j, 0))],
        out_specs=pl.BlockSpec((1, D, TILES_PER_CELL * 128),
                               lambda i, j: (i, 0, j)),
        out_shape=jax.ShapeDtypeStruct((b, D, s), jnp.float32),
    )(sel, pairs)


def kernel(x, emb_table, Wp, bp, Wj, bj, property_table):
    b, s = x.shape
    n_pair = (b * s) // 2
    pair_table, pidx = _build_prologue(
        emb_table, property_table, Wp, bp, Wj, bj, x.astype(jnp.int32))
    pairs = _make_gather(n_pair)(pair_table, pidx.reshape(n_pair // 128, 128))
    out_t = _unpack(pairs, b, s)
    return out_t.transpose(0, 2, 1)


# speedup vs baseline: 1.3380x; 1.0012x over previous
import functools

import jax
import jax.numpy as jnp
from jax import lax
from jax.experimental import pallas as pl
from jax.experimental.pallas import tpu as pltpu
from jax.experimental.pallas import tpu_sc as plsc

D = 64
VOCAB = 25
NPAIR = 640
NW = 32
CHUNK = 128
K = 2


def _pair_table_body(emb_ref, pt_ref, wp_ref, bp_ref, wj_ref, bj_ref,
                     x_ref, out_ref, idx_ref):
    b, s = x_ref.shape
    x3 = x_ref[...].reshape(b, s // 128, 128)
    pid = x3[:, :, :D] * VOCAB + x3[:, :, D:]
    idx_ref[...] = pid.reshape((b * s) // 128, D)
    rt = (((1,), (1,)), ((), ()))
    prop = lax.dot_general(pt_ref[...], wp_ref[...], rt,
                           preferred_element_type=jnp.float32) + bp_ref[...]
    combined = jnp.concatenate([emb_ref[...], prop], axis=-1)
    fused = lax.dot_general(combined, wj_ref[...], rt,
                            preferred_element_type=jnp.float32) + bj_ref[...]
    p = lax.broadcasted_iota(jnp.int32, (NPAIR, VOCAB), 0)
    v = lax.broadcasted_iota(jnp.int32, (NPAIR, VOCAB), 1)
    left = (p // VOCAB == v).astype(jnp.float32)
    right = (p % VOCAB == v).astype(jnp.float32)
    out_ref[...] = jnp.concatenate(
        [jnp.dot(left, fused, preferred_element_type=jnp.float32,
                 precision=lax.Precision.HIGHEST),
         jnp.dot(right, fused, preferred_element_type=jnp.float32,
                 precision=lax.Precision.HIGHEST)], axis=-1)


def _build_prologue(emb_table, property_table, Wp, bp, Wj, bj, x):
    b, s = x.shape
    return pl.pallas_call(
        _pair_table_body,
        out_shape=(jax.ShapeDtypeStruct((NPAIR, 2 * D), jnp.float32),
                   jax.ShapeDtypeStruct(((b * s) // 128, D), jnp.int32)),
    )(emb_table, property_table, Wp, bp.reshape(1, D), Wj,
      bj.reshape(1, D), x)


@functools.cache
def _make_gather(n_pair):
    per_w = n_pair // NW
    n_chunks = per_w // CHUNK
    n_phases = n_chunks // K
    mesh = plsc.VectorSubcoreMesh(core_axis_name="c", subcore_axis_name="s")

    @functools.partial(
        pl.kernel, mesh=mesh,
        compiler_params=pltpu.CompilerParams(use_tc_tiling_on_sc=True),
        out_type=jax.ShapeDtypeStruct((n_pair, 2 * D), jnp.float32),
        scratch_types=[
            pltpu.VMEM((n_chunks, CHUNK), jnp.int32),
            pltpu.VMEM((2, K, CHUNK, 2 * D), jnp.float32),
            pltpu.VMEM_SHARED((NPAIR, 2 * D), jnp.float32),
            pltpu.SemaphoreType.DMA,
            pltpu.SemaphoreType.DMA,
            pltpu.SemaphoreType.DMA,
            pltpu.SemaphoreType.DMA,
        ],
    )
    def gather(table_hbm, idx_hbm, out_hbm, idx_v, rows_v, table_sh,
               g0, g1, o0, o1):
        wid = lax.axis_index("s") * 2 + lax.axis_index("c")
        base = wid * per_w
        @pl.when(lax.axis_index("s") == 0)
        def _():
            pltpu.sync_copy(table_hbm, table_sh)

        pltpu.sync_copy(idx_hbm.at[pl.ds(wid * n_chunks, n_chunks), :], idx_v)
        plsc.subcore_barrier()
        gsems = (g0, g1)
        osems = (o0, o1)

        def g_copy(p, h, c):
            j = p * K + c
            return pltpu.make_async_copy(
                table_sh.at[idx_v.at[j]], rows_v.at[h].at[c], gsems[h])

        def o_copy(p, h, c):
            j = p * K + c
            return pltpu.make_async_copy(
                rows_v.at[h].at[c],
                out_hbm.at[pl.ds(base + j * CHUNK, CHUNK), :], osems[h])

        def fire_g(p, h):
            for c in range(K):
                g_copy(p, h, c).start()

        def wait_g(p, h):
            for c in range(K):
                g_copy(p, h, c).wait()

        def fire_o(p, h):
            for c in range(K):
                o_copy(p, h, c).start()

        def wait_o(p, h):
            for c in range(K):
                o_copy(p, h, c).wait()

        fire_g(0, 0)
        wait_g(0, 0)
        fire_o(0, 0)
        fire_g(1, 1)

        def body(i, carry):
            p0 = 2 * i + 1
            wait_g(p0, 1)
            fire_o(p0, 1)
            wait_o(p0 - 1, 0)
            fire_g(p0 + 1, 0)
            p1 = p0 + 1
            wait_g(p1, 0)
            fire_o(p1, 0)
            wait_o(p1 - 1, 1)
            fire_g(p1 + 1, 1)
            return carry

        lax.fori_loop(0, (n_phases - 2) // 2, body, 0)

        p = n_phases - 1
        wait_g(p, 1)
        fire_o(p, 1)
        wait_o(p - 1, 0)
        wait_o(p, 1)

    return gather


def _unpack_body(sel_ref, in_ref, out_ref):
    sel = sel_ref[...]

    def split(a):
        hi = a.astype(jnp.bfloat16)
        lo = (a - hi.astype(jnp.float32)).astype(jnp.bfloat16)
        return hi, lo

    for t in range(TILES_PER_CELL):
        blk = in_ref[pl.ds(t * D, D), :]
        a_hi, a_lo = split(blk[:, :D])
        b_hi, b_lo = split(blk[:, D:])
        m = jnp.concatenate([a_hi, a_lo, b_hi, b_lo], axis=0)
        out_ref[0, :, pl.ds(t * 128, 128)] = lax.dot_general(
            m, sel, (((0,), (0,)), ((), ())),
            preferred_element_type=jnp.float32)


TILES_PER_CELL = 16


def _unpack(pairs, b, s):
    eye = jnp.eye(D, dtype=jnp.bfloat16)
    zero = jnp.zeros((D, D), dtype=jnp.bfloat16)
    left = jnp.concatenate([eye, eye, zero, zero], axis=0)
    right = jnp.concatenate([zero, zero, eye, eye], axis=0)
    sel = jnp.concatenate([left, right], axis=1)
    scells = (s // 128) // TILES_PER_CELL
    return pl.pallas_call(
        _unpack_body,
        grid=(b, scells),
        compiler_params=pltpu.CompilerParams(
            fuse_transposed_lhs_in_matmul=True),
        in_specs=[pl.BlockSpec((4 * D, 2 * D), lambda i, j: (0, 0)),
                  pl.BlockSpec((TILES_PER_CELL * D, 2 * D),
                               lambda i, j: (i * scells + j, 0))],
        out_specs=pl.BlockSpec((1, D, TILES_PER_CELL * 128),
                               lambda i, j: (i, 0, j)),
        out_shape=jax.ShapeDtypeStruct((b, D, s), jnp.float32),
    )(sel, pairs)


def kernel(x, emb_table, Wp, bp, Wj, bj, property_table):
    b, s = x.shape
    n_pair = (b * s) // 2
    pair_table, pidx = _build_prologue(
        emb_table, property_table, Wp, bp, Wj, bj, x.astype(jnp.int32))
    pairs = _make_gather(n_pair)(pair_table, pidx.reshape(n_pair // 128, 128))
    out_t = _unpack(pairs, b, s)
    return out_t.transpose(0, 2, 1)
